# baseline (speedup 1.0000x reference)
import functools

import jax
import jax.numpy as jnp
from jax import lax
from jax.experimental import pallas as pl
from jax.experimental.pallas import tpu as pltpu
from jax.experimental.pallas import tpu_sc as plsc

N = 10000
E = 160000
D = 128
NTILE = 16
RPT = 632
NP = NTILE * RPT
CHUNK = 128
K = 80
KH = 40
EP = NTILE * K * CHUNK
ROWB = 2000


def _cnt_body(dst_st, z128, ones_in, out_c, idx_d, ones_v, cacc, sem):
    c = lax.axis_index("c")
    s = lax.axis_index("s")
    r0 = s * RPT
    pltpu.sync_copy(z128.at[pl.ds(r0, RPT)], cacc.at[pl.ds(r0, RPT)])
    pltpu.sync_copy(dst_st.at[c, s], idx_d)
    pltpu.sync_copy(ones_in, ones_v)
    plsc.subcore_barrier()

    def step(j, carry):
        pltpu.async_copy(ones_v, cacc.at[idx_d.at[j]], sem, add=True)
        return carry

    lax.fori_loop(0, K, step, 0)

    def drain(j, carry):
        pltpu.make_async_copy(ones_v, cacc.at[pl.ds(0, CHUNK)], sem).wait()
        return carry

    lax.fori_loop(0, K, drain, 0)
    plsc.subcore_barrier()
    pltpu.sync_copy(cacc.at[pl.ds(r0, RPT)], out_c.at[c, pl.ds(r0, RPT)])


def _seg_body(table, src_st, dst_st, z128,
              out_s, idx_s, idx_d, rows0, rows1, acc, sem0, sem1):
    c = lax.axis_index("c")
    s = lax.axis_index("s")
    r0 = s * RPT
    pltpu.sync_copy(z128.at[pl.ds(r0, RPT)], acc.at[pl.ds(r0, RPT)])
    plsc.subcore_barrier()

    rows = (rows0, rows1)
    sems = (sem0, sem1)
    for p in range(K // KH):
        pltpu.sync_copy(src_st.at[c, s, pl.ds(p * KH, KH)], idx_s)
        pltpu.sync_copy(dst_st.at[c, s, pl.ds(p * KH, KH)], idx_d)
        pltpu.async_copy(table.at[idx_s.at[0]], rows0, sem0)
        pltpu.async_copy(table.at[idx_s.at[1]], rows1, sem1)

        @pl.loop(0, KH, step=2)
        def _(j0):
            for b in range(2):
                pltpu.make_async_copy(table.at[pl.ds(0, CHUNK)],
                                      rows[b], sems[b]).wait()
                pltpu.sync_copy(rows[b], acc.at[idx_d.at[j0 + b]], add=True)

                @pl.when(j0 + b + 2 < KH)
                def _():
                    pltpu.async_copy(table.at[idx_s.at[j0 + b + 2]],
                                     rows[b], sems[b])

    plsc.subcore_barrier()
    pltpu.sync_copy(acc.at[pl.ds(r0, RPT)], out_s.at[c, pl.ds(r0, RPT)])


def _sc_mesh():
    return plsc.VectorSubcoreMesh(core_axis_name="c", subcore_axis_name="s")


@jax.jit
def _counts(dst_st, z128, ones_in):
    f32 = jnp.float32
    return pl.kernel(
        _cnt_body,
        out_type=jax.ShapeDtypeStruct((2, NP, D), f32),
        mesh=_sc_mesh(),
        scratch_types=[
            pltpu.VMEM((K, CHUNK), jnp.int32),
            pltpu.VMEM((CHUNK, D), f32),
            pltpu.VMEM_SHARED((NP, D), f32),
            pltpu.SemaphoreType.DMA,
        ],
    )(dst_st, z128, ones_in)


@jax.jit
def _segsum(table, src_st, dst_st, z128):
    f32 = jnp.float32
    return pl.kernel(
        _seg_body,
        out_type=jax.ShapeDtypeStruct((2, NP, D), f32),
        mesh=_sc_mesh(),
        scratch_types=[
            pltpu.VMEM((KH, CHUNK), jnp.int32),
            pltpu.VMEM((KH, CHUNK), jnp.int32),
            pltpu.VMEM((CHUNK, D), f32),
            pltpu.VMEM((CHUNK, D), f32),
            pltpu.VMEM_SHARED((NP, D), f32),
            pltpu.SemaphoreType.DMA,
            pltpu.SemaphoreType.DMA,
        ],
    )(table, src_st, dst_st, z128)


def _mlp_body(x_ref, w_ref, b_ref, o_ref):
    o_ref[0] = jnp.dot(x_ref[0], w_ref[0],
                       preferred_element_type=jnp.float32) + b_ref[0]


def _mlp2(x_st, w_st, b_st):
    g, n, d_in = x_st.shape
    d_out = w_st.shape[2]
    return pl.pallas_call(
        _mlp_body,
        grid=(g, n // ROWB),
        in_specs=[
            pl.BlockSpec((1, ROWB, d_in), lambda g, i: (g, i, 0)),
            pl.BlockSpec((1, d_in, d_out), lambda g, i: (g, 0, 0)),
            pl.BlockSpec((1, 1, d_out), lambda g, i: (g, 0, 0)),
        ],
        out_specs=pl.BlockSpec((1, ROWB, d_out), lambda g, i: (g, i, 0)),
        out_shape=jax.ShapeDtypeStruct((g, n, d_out), jnp.float32),
    )(x_st, w_st, b_st)


def _sage_post_body(s_ref, c_ref, x_ref, wl_ref, bl_ref, wr_ref, o_ref):
    inv = 1.0 / jnp.maximum(c_ref[0][:, 0:1], 1.0)
    mean = s_ref[0] * inv
    t = (jnp.dot(mean, wl_ref[0], preferred_element_type=jnp.float32)
         + bl_ref[0]
         + jnp.dot(x_ref[0], wr_ref[0], preferred_element_type=jnp.float32))
    nrm = jnp.sqrt(jnp.sum(t * t, axis=1, keepdims=True))
    t = t / jnp.maximum(nrm, 1e-12)
    o_ref[0] = jnp.maximum(t, 0.0)


def _sage_post2(s_st, c_st, x_st, wl_st, bl_st, wr_st):
    d_out = wl_st.shape[2]
    return pl.pallas_call(
        _sage_post_body,
        grid=(2, N // ROWB),
        in_specs=[
            pl.BlockSpec((1, ROWB, D), lambda g, i: (g, i, 0)),
            pl.BlockSpec((1, ROWB, 16), lambda g, i: (g, i, 0)),
            pl.BlockSpec((1, ROWB, D), lambda g, i: (1 - g, i, 0)),
            pl.BlockSpec((1, D, d_out), lambda g, i: (g, 0, 0)),
            pl.BlockSpec((1, 1, d_out), lambda g, i: (g, 0, 0)),
            pl.BlockSpec((1, D, d_out), lambda g, i: (g, 0, 0)),
        ],
        out_specs=pl.BlockSpec((1, ROWB, d_out), lambda g, i: (g, i, 0)),
        out_shape=jax.ShapeDtypeStruct((2, N, d_out), jnp.float32),
    )(s_st, c_st, x_st, wl_st, bl_st, wr_st)


def _final_body(slo_ref, shi_ref, c_ref, x_ref, wlo_ref, whi_ref, bl_ref,
                wr_ref, lw_ref, lb_ref, o_ref):
    inv = 1.0 / jnp.maximum(c_ref[...][:, 0:1], 1.0)
    t = (jnp.dot(slo_ref[...] * inv, wlo_ref[...],
                 preferred_element_type=jnp.float32)
         + jnp.dot(shi_ref[...] * inv, whi_ref[...],
                   preferred_element_type=jnp.float32)
         + bl_ref[...]
         + jnp.dot(x_ref[...], wr_ref[...], preferred_element_type=jnp.float32))
    nrm = jnp.sqrt(jnp.sum(t * t, axis=1, keepdims=True))
    t = t / jnp.maximum(nrm, 1e-12)
    t = jnp.maximum(t, 0.0)
    o_ref[...] = jnp.dot(t, lw_ref[...],
                         preferred_element_type=jnp.float32) + lb_ref[...]


def _final(s_lo, s_hi, cnt, x, wl_lo, wl_hi, bl, wr, lw, lb):
    n = s_lo.shape[0]
    dh = wr.shape[1]
    do = lw.shape[1]
    return pl.pallas_call(
        _final_body,
        grid=(n // ROWB,),
        in_specs=[
            pl.BlockSpec((ROWB, D), lambda i: (i, 0)),
            pl.BlockSpec((ROWB, D), lambda i: (i, 0)),
            pl.BlockSpec((ROWB, 16), lambda i: (i, 0)),
            pl.BlockSpec((ROWB, dh), lambda i: (i, 0)),
            pl.BlockSpec((D, dh), lambda i: (0, 0)),
            pl.BlockSpec((D, dh), lambda i: (0, 0)),
            pl.BlockSpec((1, dh), lambda i: (0, 0)),
            pl.BlockSpec((dh, dh), lambda i: (0, 0)),
            pl.BlockSpec((dh, do), lambda i: (0, 0)),
            pl.BlockSpec((1, do), lambda i: (0, 0)),
        ],
        out_specs=pl.BlockSpec((ROWB, do), lambda i: (i, 0)),
        out_shape=jax.ShapeDtypeStruct((n, do), jnp.float32),
    )(s_lo, s_hi, cnt, x, wl_lo, wl_hi, bl, wr, lw, lb)


def _pad_edges(src, dst):
    pad = EP - E
    src_p = jnp.concatenate([src, jnp.zeros((pad,), jnp.int32)])
    dst_p = jnp.concatenate([dst, jnp.full((pad,), N, jnp.int32)])
    return src_p, dst_p


def kernel(x_user, x_item, edge_index_u2i, edge_index_i2u,
           mlp_user_W, mlp_user_b, mlp_item_W, mlp_item_b,
           l0_u2i_Wl, l0_u2i_bl, l0_u2i_Wr,
           l0_i2u_Wl, l0_i2u_bl, l0_i2u_Wr,
           l1_u2i_Wl, l1_u2i_bl, l1_u2i_Wr,
           l1_i2u_Wl, l1_i2u_bl, l1_i2u_Wr,
           lin_W, lin_b):
    f32 = jnp.float32
    ei_u2i = edge_index_u2i.astype(jnp.int32)
    ei_i2u = edge_index_i2u.astype(jnp.int32)
    su, du = _pad_edges(ei_u2i[0], ei_u2i[1])
    si, di = _pad_edges(ei_i2u[0], ei_i2u[1])

    z128 = jnp.zeros((NP, D), f32)
    ones128 = jnp.ones((CHUNK, D), f32)

    h_st = _mlp2(jnp.stack([x_user, x_item]),
                 jnp.stack([mlp_user_W, mlp_item_W]),
                 jnp.stack([mlp_user_b.reshape(1, -1),
                            mlp_item_b.reshape(1, -1)]))

    dst_st0 = jnp.stack([du, di]).reshape(2, NTILE, K, CHUNK)
    out_c0 = _counts(dst_st0, z128, ones128)

    table0 = h_st.reshape(2 * N, D)
    src_st0 = jnp.stack([su, si + N]).reshape(2, NTILE, K, CHUNK)
    out_s0 = _segsum(table0, src_st0, dst_st0, z128)

    hio_st = _sage_post2(
        out_s0, out_c0[:, :, :16], h_st,
        jnp.stack([l0_u2i_Wl, l0_i2u_Wl]),
        jnp.stack([l0_u2i_bl.reshape(1, -1), l0_i2u_bl.reshape(1, -1)]),
        jnp.stack([l0_u2i_Wr, l0_i2u_Wr]))
    hi1, hu1 = hio_st[0], hio_st[1]
    cnt_u = out_c0[1, :N, :16]

    table1 = jnp.concatenate([hi1[:, :D], hi1[:, D:]], axis=0)
    src_st1 = jnp.stack([si, si + N]).reshape(2, NTILE, K, CHUNK)
    dst_st1 = jnp.stack([di, di]).reshape(2, NTILE, K, CHUNK)
    out_s1 = _segsum(table1, src_st1, dst_st1, z128)
    s_lo, s_hi = out_s1[0, :N], out_s1[1, :N]

    return _final(s_lo, s_hi, cnt_u, hu1,
                  l1_i2u_Wl[:D], l1_i2u_Wl[D:], l1_i2u_bl.reshape(1, -1),
                  l1_i2u_Wr, lin_W, lin_b.reshape(1, -1))


# speedup vs baseline: 1.0443x; 1.0443x over previous
import functools

import jax
import jax.numpy as jnp
from jax import lax
from jax.experimental import pallas as pl
from jax.experimental.pallas import tpu as pltpu
from jax.experimental.pallas import tpu_sc as plsc

N = 10000
E = 160000
D = 128
NTILE = 16
RPT = 632
NP = NTILE * RPT
CHUNK = 128
K = 80
KH = 40
EP = NTILE * K * CHUNK
ROWB = 2000


def _cnt_body(dst_st, z128, ones_in, out_c, idx_d, ones_v, cacc, sem):
    c = lax.axis_index("c")
    s = lax.axis_index("s")
    r0 = s * RPT
    pltpu.sync_copy(z128.at[pl.ds(r0, RPT)], cacc.at[pl.ds(r0, RPT)])
    pltpu.sync_copy(dst_st.at[c, s], idx_d)
    pltpu.sync_copy(ones_in, ones_v)
    plsc.subcore_barrier()

    def step(j, carry):
        pltpu.async_copy(ones_v, cacc.at[idx_d.at[j]], sem, add=True)
        return carry

    lax.fori_loop(0, K, step, 0)

    def drain(j, carry):
        pltpu.make_async_copy(ones_v, cacc.at[pl.ds(0, CHUNK)], sem).wait()
        return carry

    lax.fori_loop(0, K, drain, 0)
    plsc.subcore_barrier()
    pltpu.sync_copy(cacc.at[pl.ds(r0, RPT)], out_c.at[c, pl.ds(r0, RPT)])


def _seg_body(table, src_st, dst_st, z128,
              out_s, idx_s, idx_d, rows0, rows1, acc, sem0, sem1):
    c = lax.axis_index("c")
    s = lax.axis_index("s")
    r0 = s * RPT
    pltpu.sync_copy(z128.at[pl.ds(r0, RPT)], acc.at[pl.ds(r0, RPT)])
    plsc.subcore_barrier()

    rows = (rows0, rows1)
    sems = (sem0, sem1)
    for p in range(K // KH):
        pltpu.sync_copy(src_st.at[c, s, pl.ds(p * KH, KH)], idx_s)
        pltpu.sync_copy(dst_st.at[c, s, pl.ds(p * KH, KH)], idx_d)
        pltpu.async_copy(table.at[idx_s.at[0]], rows0, sem0)
        pltpu.async_copy(table.at[idx_s.at[1]], rows1, sem1)

        @pl.loop(0, KH, step=2)
        def _(j0):
            for b in range(2):
                pltpu.make_async_copy(table.at[pl.ds(0, CHUNK)],
                                      rows[b], sems[b]).wait()
                pltpu.sync_copy(rows[b], acc.at[idx_d.at[j0 + b]], add=True)

                @pl.when(j0 + b + 2 < KH)
                def _():
                    pltpu.async_copy(table.at[idx_s.at[j0 + b + 2]],
                                     rows[b], sems[b])

    plsc.subcore_barrier()
    pltpu.sync_copy(acc.at[pl.ds(r0, RPT)], out_s.at[c, pl.ds(r0, RPT)])


def _sc_mesh():
    return plsc.VectorSubcoreMesh(core_axis_name="c", subcore_axis_name="s")


@jax.jit
def _counts(dst_st, z128, ones_in):
    f32 = jnp.float32
    return pl.kernel(
        _cnt_body,
        out_type=jax.ShapeDtypeStruct((2, NP, D), f32),
        mesh=_sc_mesh(),
        scratch_types=[
            pltpu.VMEM((K, CHUNK), jnp.int32),
            pltpu.VMEM((CHUNK, D), f32),
            pltpu.VMEM_SHARED((NP, D), f32),
            pltpu.SemaphoreType.DMA,
        ],
    )(dst_st, z128, ones_in)


@jax.jit
def _segsum(table, src_st, dst_st, z128):
    f32 = jnp.float32
    return pl.kernel(
        _seg_body,
        out_type=jax.ShapeDtypeStruct((2, NP, D), f32),
        mesh=_sc_mesh(),
        scratch_types=[
            pltpu.VMEM((KH, CHUNK), jnp.int32),
            pltpu.VMEM((KH, CHUNK), jnp.int32),
            pltpu.VMEM((CHUNK, D), f32),
            pltpu.VMEM((CHUNK, D), f32),
            pltpu.VMEM_SHARED((NP, D), f32),
            pltpu.SemaphoreType.DMA,
            pltpu.SemaphoreType.DMA,
        ],
    )(table, src_st, dst_st, z128)


def _mlp_body(x_ref, w_ref, b_ref, o_ref):
    o_ref[...] = jnp.dot(x_ref[...], w_ref[...],
                         preferred_element_type=jnp.float32) + b_ref[...]


def _mlp(x, w, b):
    n, d_in = x.shape
    d_out = w.shape[1]
    return pl.pallas_call(
        _mlp_body,
        grid=(n // ROWB,),
        in_specs=[
            pl.BlockSpec((ROWB, d_in), lambda i: (i, 0)),
            pl.BlockSpec((d_in, d_out), lambda i: (0, 0)),
            pl.BlockSpec((1, d_out), lambda i: (0, 0)),
        ],
        out_specs=pl.BlockSpec((ROWB, d_out), lambda i: (i, 0)),
        out_shape=jax.ShapeDtypeStruct((n, d_out), jnp.float32),
    )(x, w, b)


def _sage_post_body(s_ref, c_ref, x_ref, wl_ref, bl_ref, wr_ref, o_ref):
    inv = 1.0 / jnp.maximum(c_ref[...][:, 0:1], 1.0)
    mean = s_ref[...] * inv
    t = (jnp.dot(mean, wl_ref[...], preferred_element_type=jnp.float32)
         + bl_ref[...]
         + jnp.dot(x_ref[...], wr_ref[...], preferred_element_type=jnp.float32))
    nrm = jnp.sqrt(jnp.sum(t * t, axis=1, keepdims=True))
    t = t / jnp.maximum(nrm, 1e-12)
    o_ref[...] = jnp.maximum(t, 0.0)


def _sage_post(s, cnt, x, wl, bl, wr):
    n, d_in = s.shape
    d_out = wl.shape[1]
    d_x = x.shape[1]
    return pl.pallas_call(
        _sage_post_body,
        grid=(n // ROWB,),
        in_specs=[
            pl.BlockSpec((ROWB, d_in), lambda i: (i, 0)),
            pl.BlockSpec((ROWB, 16), lambda i: (i, 0)),
            pl.BlockSpec((ROWB, d_x), lambda i: (i, 0)),
            pl.BlockSpec((d_in, d_out), lambda i: (0, 0)),
            pl.BlockSpec((1, d_out), lambda i: (0, 0)),
            pl.BlockSpec((d_x, d_out), lambda i: (0, 0)),
        ],
        out_specs=pl.BlockSpec((ROWB, d_out), lambda i: (i, 0)),
        out_shape=jax.ShapeDtypeStruct((n, d_out), jnp.float32),
    )(s, cnt, x, wl, bl, wr)


def _final_body(slo_ref, shi_ref, c_ref, x_ref, wlo_ref, whi_ref, bl_ref,
                wr_ref, lw_ref, lb_ref, o_ref):
    inv = 1.0 / jnp.maximum(c_ref[...][:, 0:1], 1.0)
    t = (jnp.dot(slo_ref[...] * inv, wlo_ref[...],
                 preferred_element_type=jnp.float32)
         + jnp.dot(shi_ref[...] * inv, whi_ref[...],
                   preferred_element_type=jnp.float32)
         + bl_ref[...]
         + jnp.dot(x_ref[...], wr_ref[...], preferred_element_type=jnp.float32))
    nrm = jnp.sqrt(jnp.sum(t * t, axis=1, keepdims=True))
    t = t / jnp.maximum(nrm, 1e-12)
    t = jnp.maximum(t, 0.0)
    o_ref[...] = jnp.dot(t, lw_ref[...],
                         preferred_element_type=jnp.float32) + lb_ref[...]


def _final(s_lo, s_hi, cnt, x, wl_lo, wl_hi, bl, wr, lw, lb):
    n = s_lo.shape[0]
    dh = wr.shape[1]
    do = lw.shape[1]
    return pl.pallas_call(
        _final_body,
        grid=(n // ROWB,),
        in_specs=[
            pl.BlockSpec((ROWB, D), lambda i: (i, 0)),
            pl.BlockSpec((ROWB, D), lambda i: (i, 0)),
            pl.BlockSpec((ROWB, 16), lambda i: (i, 0)),
            pl.BlockSpec((ROWB, dh), lambda i: (i, 0)),
            pl.BlockSpec((D, dh), lambda i: (0, 0)),
            pl.BlockSpec((D, dh), lambda i: (0, 0)),
            pl.BlockSpec((1, dh), lambda i: (0, 0)),
            pl.BlockSpec((dh, dh), lambda i: (0, 0)),
            pl.BlockSpec((dh, do), lambda i: (0, 0)),
            pl.BlockSpec((1, do), lambda i: (0, 0)),
        ],
        out_specs=pl.BlockSpec((ROWB, do), lambda i: (i, 0)),
        out_shape=jax.ShapeDtypeStruct((n, do), jnp.float32),
    )(s_lo, s_hi, cnt, x, wl_lo, wl_hi, bl, wr, lw, lb)


def _pad_edges(src, dst):
    pad = EP - E
    src_p = jnp.concatenate([src, jnp.zeros((pad,), jnp.int32)])
    dst_p = jnp.concatenate([dst, jnp.full((pad,), N, jnp.int32)])
    return src_p, dst_p


def kernel(x_user, x_item, edge_index_u2i, edge_index_i2u,
           mlp_user_W, mlp_user_b, mlp_item_W, mlp_item_b,
           l0_u2i_Wl, l0_u2i_bl, l0_u2i_Wr,
           l0_i2u_Wl, l0_i2u_bl, l0_i2u_Wr,
           l1_u2i_Wl, l1_u2i_bl, l1_u2i_Wr,
           l1_i2u_Wl, l1_i2u_bl, l1_i2u_Wr,
           lin_W, lin_b):
    f32 = jnp.float32
    ei_u2i = edge_index_u2i.astype(jnp.int32)
    ei_i2u = edge_index_i2u.astype(jnp.int32)
    su, du = _pad_edges(ei_u2i[0], ei_u2i[1])
    si, di = _pad_edges(ei_i2u[0], ei_i2u[1])

    z128 = jnp.zeros((NP, D), f32)
    ones128 = jnp.ones((CHUNK, D), f32)

    hu = _mlp(x_user, mlp_user_W, mlp_user_b.reshape(1, -1))
    hi = _mlp(x_item, mlp_item_W, mlp_item_b.reshape(1, -1))

    dst_st0 = jnp.stack([du, di]).reshape(2, NTILE, K, CHUNK)
    out_c0 = _counts(dst_st0, z128, ones128)
    cnt_i, cnt_u = out_c0[0, :N, :16], out_c0[1, :N, :16]

    table0 = jnp.concatenate([hu, hi], axis=0)
    src_st0 = jnp.stack([su, si + N]).reshape(2, NTILE, K, CHUNK)
    out_s0 = _segsum(table0, src_st0, dst_st0, z128)
    s_i, s_u = out_s0[0, :N], out_s0[1, :N]

    hi1 = _sage_post(s_i, cnt_i, hi, l0_u2i_Wl, l0_u2i_bl.reshape(1, -1),
                     l0_u2i_Wr)
    hu1 = _sage_post(s_u, cnt_u, hu, l0_i2u_Wl, l0_i2u_bl.reshape(1, -1),
                     l0_i2u_Wr)

    table1 = jnp.concatenate([hi1[:, :D], hi1[:, D:]], axis=0)
    src_st1 = jnp.stack([si, si + N]).reshape(2, NTILE, K, CHUNK)
    dst_st1 = jnp.stack([di, di]).reshape(2, NTILE, K, CHUNK)
    out_s1 = _segsum(table1, src_st1, dst_st1, z128)
    s_lo, s_hi = out_s1[0, :N], out_s1[1, :N]

    return _final(s_lo, s_hi, cnt_u, hu1,
                  l1_i2u_Wl[:D], l1_i2u_Wl[D:], l1_i2u_bl.reshape(1, -1),
                  l1_i2u_Wr, lin_W, lin_b.reshape(1, -1))


# speedup vs baseline: 1.0716x; 1.0261x over previous
import functools

import jax
import jax.numpy as jnp
from jax import lax
from jax.experimental import pallas as pl
from jax.experimental.pallas import tpu as pltpu
from jax.experimental.pallas import tpu_sc as plsc

N = 10000
E = 160000
D = 128
NTILE = 16
RPT = 632
NP = NTILE * RPT
CHUNK = 64
K = 160
KH = 40
NBUF = 4
EP = NTILE * K * CHUNK
ROWB = 2000


def _cnt_body(dst_st, z128, ones_in, out_c, idx_d, ones_v, cacc, sem):
    c = lax.axis_index("c")
    s = lax.axis_index("s")
    r0 = s * RPT
    pltpu.sync_copy(z128.at[pl.ds(r0, RPT)], cacc.at[pl.ds(r0, RPT)])
    pltpu.sync_copy(dst_st.at[c, s], idx_d)
    pltpu.sync_copy(ones_in, ones_v)
    plsc.subcore_barrier()

    def step(j, carry):
        pltpu.async_copy(ones_v, cacc.at[idx_d.at[j]], sem, add=True)
        return carry

    lax.fori_loop(0, K, step, 0)

    def drain(j, carry):
        pltpu.make_async_copy(ones_v, cacc.at[pl.ds(0, CHUNK)], sem).wait()
        return carry

    lax.fori_loop(0, K, drain, 0)
    plsc.subcore_barrier()
    pltpu.sync_copy(cacc.at[pl.ds(r0, RPT)], out_c.at[c, pl.ds(r0, RPT)])


def _seg_body(table, src_st, dst_st, z128,
              out_s, idx_s, idx_d, rows0, rows1, rows2, rows3, acc,
              sem0, sem1, sem2, sem3):
    c = lax.axis_index("c")
    s = lax.axis_index("s")
    r0 = s * RPT
    pltpu.sync_copy(z128.at[pl.ds(r0, RPT)], acc.at[pl.ds(r0, RPT)])
    plsc.subcore_barrier()

    rows = (rows0, rows1, rows2, rows3)
    sems = (sem0, sem1, sem2, sem3)
    for p in range(K // KH):
        pltpu.sync_copy(src_st.at[c, s, pl.ds(p * KH, KH)], idx_s)
        pltpu.sync_copy(dst_st.at[c, s, pl.ds(p * KH, KH)], idx_d)
        for b in range(NBUF):
            pltpu.async_copy(table.at[idx_s.at[b]], rows[b], sems[b])

        @pl.loop(0, KH, step=NBUF)
        def _(j0):
            for b in range(NBUF):
                pltpu.make_async_copy(table.at[pl.ds(0, CHUNK)],
                                      rows[b], sems[b]).wait()
                pltpu.sync_copy(rows[b], acc.at[idx_d.at[j0 + b]], add=True)

                @pl.when(j0 + b + NBUF < KH)
                def _():
                    pltpu.async_copy(table.at[idx_s.at[j0 + b + NBUF]],
                                     rows[b], sems[b])

    plsc.subcore_barrier()
    pltpu.sync_copy(acc.at[pl.ds(r0, RPT)], out_s.at[c, pl.ds(r0, RPT)])


def _sc_mesh():
    return plsc.VectorSubcoreMesh(core_axis_name="c", subcore_axis_name="s")


@jax.jit
def _counts(dst_st, z128, ones_in):
    f32 = jnp.float32
    return pl.kernel(
        _cnt_body,
        out_type=jax.ShapeDtypeStruct((2, NP, D), f32),
        mesh=_sc_mesh(),
        scratch_types=[
            pltpu.VMEM((K, CHUNK), jnp.int32),
            pltpu.VMEM((CHUNK, D), f32),
            pltpu.VMEM_SHARED((NP, D), f32),
            pltpu.SemaphoreType.DMA,
        ],
    )(dst_st, z128, ones_in)


@jax.jit
def _segsum(table, src_st, dst_st, z128):
    f32 = jnp.float32
    return pl.kernel(
        _seg_body,
        out_type=jax.ShapeDtypeStruct((2, NP, D), f32),
        mesh=_sc_mesh(),
        scratch_types=[
            pltpu.VMEM((KH, CHUNK), jnp.int32),
            pltpu.VMEM((KH, CHUNK), jnp.int32),
            pltpu.VMEM((CHUNK, D), f32),
            pltpu.VMEM((CHUNK, D), f32),
            pltpu.VMEM((CHUNK, D), f32),
            pltpu.VMEM((CHUNK, D), f32),
            pltpu.VMEM_SHARED((NP, D), f32),
            pltpu.SemaphoreType.DMA,
            pltpu.SemaphoreType.DMA,
            pltpu.SemaphoreType.DMA,
            pltpu.SemaphoreType.DMA,
        ],
    )(table, src_st, dst_st, z128)


def _mlp_body(x_ref, w_ref, b_ref, o_ref):
    o_ref[...] = jnp.dot(x_ref[...], w_ref[...],
                         preferred_element_type=jnp.float32) + b_ref[...]


def _mlp(x, w, b):
    n, d_in = x.shape
    d_out = w.shape[1]
    return pl.pallas_call(
        _mlp_body,
        grid=(n // ROWB,),
        in_specs=[
            pl.BlockSpec((ROWB, d_in), lambda i: (i, 0)),
            pl.BlockSpec((d_in, d_out), lambda i: (0, 0)),
            pl.BlockSpec((1, d_out), lambda i: (0, 0)),
        ],
        out_specs=pl.BlockSpec((ROWB, d_out), lambda i: (i, 0)),
        out_shape=jax.ShapeDtypeStruct((n, d_out), jnp.float32),
    )(x, w, b)


def _sage_post_body(s_ref, c_ref, x_ref, wl_ref, bl_ref, wr_ref, o_ref):
    inv = 1.0 / jnp.maximum(c_ref[...][:, 0:1], 1.0)
    mean = s_ref[...] * inv
    t = (jnp.dot(mean, wl_ref[...], preferred_element_type=jnp.float32)
         + bl_ref[...]
         + jnp.dot(x_ref[...], wr_ref[...], preferred_element_type=jnp.float32))
    nrm = jnp.sqrt(jnp.sum(t * t, axis=1, keepdims=True))
    t = t / jnp.maximum(nrm, 1e-12)
    o_ref[...] = jnp.maximum(t, 0.0)


def _sage_post(s, cnt, x, wl, bl, wr):
    n, d_in = s.shape
    d_out = wl.shape[1]
    d_x = x.shape[1]
    return pl.pallas_call(
        _sage_post_body,
        grid=(n // ROWB,),
        in_specs=[
            pl.BlockSpec((ROWB, d_in), lambda i: (i, 0)),
            pl.BlockSpec((ROWB, 16), lambda i: (i, 0)),
            pl.BlockSpec((ROWB, d_x), lambda i: (i, 0)),
            pl.BlockSpec((d_in, d_out), lambda i: (0, 0)),
            pl.BlockSpec((1, d_out), lambda i: (0, 0)),
            pl.BlockSpec((d_x, d_out), lambda i: (0, 0)),
        ],
        out_specs=pl.BlockSpec((ROWB, d_out), lambda i: (i, 0)),
        out_shape=jax.ShapeDtypeStruct((n, d_out), jnp.float32),
    )(s, cnt, x, wl, bl, wr)


def _final_body(slo_ref, shi_ref, c_ref, x_ref, wlo_ref, whi_ref, bl_ref,
                wr_ref, lw_ref, lb_ref, o_ref):
    inv = 1.0 / jnp.maximum(c_ref[...][:, 0:1], 1.0)
    t = (jnp.dot(slo_ref[...] * inv, wlo_ref[...],
                 preferred_element_type=jnp.float32)
         + jnp.dot(shi_ref[...] * inv, whi_ref[...],
                   preferred_element_type=jnp.float32)
         + bl_ref[...]
         + jnp.dot(x_ref[...], wr_ref[...], preferred_element_type=jnp.float32))
    nrm = jnp.sqrt(jnp.sum(t * t, axis=1, keepdims=True))
    t = t / jnp.maximum(nrm, 1e-12)
    t = jnp.maximum(t, 0.0)
    o_ref[...] = jnp.dot(t, lw_ref[...],
                         preferred_element_type=jnp.float32) + lb_ref[...]


def _final(s_lo, s_hi, cnt, x, wl_lo, wl_hi, bl, wr, lw, lb):
    n = s_lo.shape[0]
    dh = wr.shape[1]
    do = lw.shape[1]
    return pl.pallas_call(
        _final_body,
        grid=(n // ROWB,),
        in_specs=[
            pl.BlockSpec((ROWB, D), lambda i: (i, 0)),
            pl.BlockSpec((ROWB, D), lambda i: (i, 0)),
            pl.BlockSpec((ROWB, 16), lambda i: (i, 0)),
            pl.BlockSpec((ROWB, dh), lambda i: (i, 0)),
            pl.BlockSpec((D, dh), lambda i: (0, 0)),
            pl.BlockSpec((D, dh), lambda i: (0, 0)),
            pl.BlockSpec((1, dh), lambda i: (0, 0)),
            pl.BlockSpec((dh, dh), lambda i: (0, 0)),
            pl.BlockSpec((dh, do), lambda i: (0, 0)),
            pl.BlockSpec((1, do), lambda i: (0, 0)),
        ],
        out_specs=pl.BlockSpec((ROWB, do), lambda i: (i, 0)),
        out_shape=jax.ShapeDtypeStruct((n, do), jnp.float32),
    )(s_lo, s_hi, cnt, x, wl_lo, wl_hi, bl, wr, lw, lb)


def _pad_edges(src, dst):
    pad = EP - E
    src_p = jnp.concatenate([src, jnp.zeros((pad,), jnp.int32)])
    dst_p = jnp.concatenate([dst, jnp.full((pad,), N, jnp.int32)])
    return src_p, dst_p


def kernel(x_user, x_item, edge_index_u2i, edge_index_i2u,
           mlp_user_W, mlp_user_b, mlp_item_W, mlp_item_b,
           l0_u2i_Wl, l0_u2i_bl, l0_u2i_Wr,
           l0_i2u_Wl, l0_i2u_bl, l0_i2u_Wr,
           l1_u2i_Wl, l1_u2i_bl, l1_u2i_Wr,
           l1_i2u_Wl, l1_i2u_bl, l1_i2u_Wr,
           lin_W, lin_b):
    f32 = jnp.float32
    ei_u2i = edge_index_u2i.astype(jnp.int32)
    ei_i2u = edge_index_i2u.astype(jnp.int32)
    su, du = _pad_edges(ei_u2i[0], ei_u2i[1])
    si, di = _pad_edges(ei_i2u[0], ei_i2u[1])

    z128 = jnp.zeros((NP, D), f32)
    ones128 = jnp.ones((CHUNK, D), f32)

    hu = _mlp(x_user, mlp_user_W, mlp_user_b.reshape(1, -1))
    hi = _mlp(x_item, mlp_item_W, mlp_item_b.reshape(1, -1))

    dst_st0 = jnp.stack([du, di]).reshape(2, NTILE, K, CHUNK)
    out_c0 = _counts(dst_st0, z128, ones128)
    cnt_i, cnt_u = out_c0[0, :N, :16], out_c0[1, :N, :16]

    table0 = jnp.concatenate([hu, hi], axis=0)
    src_st0 = jnp.stack([su, si + N]).reshape(2, NTILE, K, CHUNK)
    out_s0 = _segsum(table0, src_st0, dst_st0, z128)
    s_i, s_u = out_s0[0, :N], out_s0[1, :N]

    hi1 = _sage_post(s_i, cnt_i, hi, l0_u2i_Wl, l0_u2i_bl.reshape(1, -1),
                     l0_u2i_Wr)
    hu1 = _sage_post(s_u, cnt_u, hu, l0_i2u_Wl, l0_i2u_bl.reshape(1, -1),
                     l0_i2u_Wr)

    table1 = jnp.concatenate([hi1[:, :D], hi1[:, D:]], axis=0)
    src_st1 = jnp.stack([si, si + N]).reshape(2, NTILE, K, CHUNK)
    dst_st1 = jnp.stack([di, di]).reshape(2, NTILE, K, CHUNK)
    out_s1 = _segsum(table1, src_st1, dst_st1, z128)
    s_lo, s_hi = out_s1[0, :N], out_s1[1, :N]

    return _final(s_lo, s_hi, cnt_u, hu1,
                  l1_i2u_Wl[:D], l1_i2u_Wl[D:], l1_i2u_bl.reshape(1, -1),
                  l1_i2u_Wr, lin_W, lin_b.reshape(1, -1))
